# baseline (speedup 1.0000x reference)
import functools

import jax
import jax.numpy as jnp
from jax import lax
from jax.experimental import pallas as pl
from jax.experimental.pallas import tpu as pltpu
import jax.experimental.pallas.tpu_sc as plsc

B = 16; T = 20; CODE = 2000; D = 128; N = 10000; E = 320000; R = 16
E_PAD = 327680
ECH = 128
NP = 10240
NB = 10
BLK = NP // NB
NR = R + 1
NPC = NP * NR
NC, NS, L = 2, 16, 16
NW = NC * NS
CH = 80
NCH2 = E_PAD // NW // ECH
ROWS_PER_TILE = NP // NS
F32 = jnp.float32
I32 = jnp.int32


def _sigmoid(x):
    return 1.0 / (1.0 + jnp.exp(-x))


def _front_body(x_ref, mask_ref, ehr_ref, wx_ref, wh_ref, b_ref, aw_ref,
                sf_ref, alpha_ref, hs_scr, h_scr, sc_scr):
    t = pl.program_id(0)

    @pl.when(t == 0)
    def _():
        h_scr[...] = jnp.zeros((B, D), F32)
        sc_scr[...] = jnp.zeros((B, D), F32)

    h = h_scr[...]
    xt = x_ref[0]
    e = jnp.dot(xt, ehr_ref[...], preferred_element_type=F32)
    gx = jnp.dot(e, wx_ref[...], preferred_element_type=F32) + b_ref[...]
    gh = jnp.dot(h, wh_ref[...], preferred_element_type=F32)
    z = _sigmoid(gx[:, :D] + gh[:, :D])
    r = _sigmoid(gx[:, D:2 * D] + gh[:, D:2 * D])
    n = jnp.tanh(gx[:, 2 * D:] + r * gh[:, 2 * D:])
    h = (1.0 - z) * n + z * h
    h_scr[...] = h
    hs_scr[t] = h
    sval = jnp.sum(jnp.tanh(h) * aw_ref[...], axis=1, keepdims=True)
    lane = lax.broadcasted_iota(I32, (B, D), 1)
    sc_scr[...] += jnp.where(lane == t, sval, 0.0)

    @pl.when(t == T - 1)
    def _():
        s = sc_scr[:, :T] + (mask_ref[...] - 1.0) * 1e9
        smax = jnp.max(s, axis=1, keepdims=True)
        ex = jnp.exp(s - smax)
        alpha = ex / jnp.sum(ex, axis=1, keepdims=True)
        alpha_ref[...] = alpha
        lane20 = lax.broadcasted_iota(I32, (B, T), 1)

        def acc(i, carry):
            a_i = jnp.sum(jnp.where(lane20 == i, alpha, 0.0), axis=1,
                          keepdims=True)
            return carry + a_i * hs_scr[i]

        sf_ref[...] = lax.fori_loop(0, T, acc, jnp.zeros((B, D), F32))


def _tc_front(x_tbc, batch_mask, ehrcode_W, gru_Wx, gru_Wh, gru_b, att_w):
    full = lambda shape: pl.BlockSpec(shape, lambda t: (0,) * len(shape))
    return pl.pallas_call(
        _front_body,
        grid=(T,),
        in_specs=[
            pl.BlockSpec((1, B, CODE), lambda t: (t, 0, 0)),
            full((B, T)), full((CODE, D)), full((D, 3 * D)),
            full((D, 3 * D)), full((1, 3 * D)), full((1, D)),
        ],
        out_specs=[full((B, D)), full((B, T))],
        out_shape=[jax.ShapeDtypeStruct((B, D), F32),
                   jax.ShapeDtypeStruct((B, T), F32)],
        scratch_shapes=[pltpu.VMEM((T, B, D), F32), pltpu.VMEM((B, D), F32),
                        pltpu.VMEM((B, D), F32)],
    )(x_tbc, batch_mask, ehrcode_W, gru_Wx, gru_Wh, gru_b, att_w)


def _edgeidx_body(src_ref, dst_ref, typ_ref, eb_ref, gidx_ref, cidx_ref,
                  i1_ref, i2_ref):
    s = src_ref[...]
    d = dst_ref[...]
    t = typ_ref[...]
    eb = eb_ref[...]
    gidx_ref[...] = t * NP + s
    cidx_ref[...] = d * NR + t
    i1_ref[...] = s * B + eb
    i2_ref[...] = d * B + eb


def _tc_edgeidx(src2d, dst2d, typ2d, eb2d):
    sh = src2d.shape
    full = pl.BlockSpec(sh, lambda: (0, 0))
    return pl.pallas_call(
        _edgeidx_body,
        in_specs=[full] * 4,
        out_specs=[full] * 4,
        out_shape=[jax.ShapeDtypeStruct(sh, I32)] * 4,
    )(src2d, dst2d, typ2d, eb2d)


def _samplers_body(x0_ref, macc_ref, n2g_ref, sf_ref, wpn_ref, wpe_ref,
                   wgn_ref, wge_ref, wr_ref,
                   pn_ref, vx_ref, ge_ref, h1n_ref, h1e_ref,
                   p0n_ref, p0e_ref, cnt_ref):
    i = pl.program_id(0)
    x0 = x0_ref[...]
    xm = x0 + macc_ref[0] + macc_ref[1]
    hg = jnp.maximum(jnp.dot(xm, wgn_ref[...], preferred_element_type=F32), 0.0)
    hge = jnp.maximum(jnp.dot(xm, wge_ref[...], preferred_element_type=F32), 0.0)
    qn = sf_ref[...] + wpn_ref[...]
    qe = sf_ref[...] + wpe_ref[...]
    dimn = (((1,), (1,)), ((), ()))
    Gn = lax.dot_general(hg, qn, dimn, preferred_element_type=F32)
    Ge = lax.dot_general(hge, qe, dimn, preferred_element_type=F32)
    ge_ref[...] = Ge
    onehot = (n2g_ref[...] == lax.broadcasted_iota(I32, (1, B), 1)).astype(F32)
    pick = jnp.sum(Gn * onehot, axis=1, keepdims=True)
    p = _sigmoid(pick)
    pn_ref[...] = p
    vx = x0 * p
    vx_ref[...] = vx
    dimp = (((0,), (0,)), ((), ()))

    @pl.when(i == 0)
    def _():
        p0n_ref[...] = jnp.zeros((B, D), F32)
        p0e_ref[...] = jnp.zeros((B, D), F32)
        cnt_ref[...] = jnp.zeros((B, D), F32)

    p0n_ref[...] += lax.dot_general(onehot, vx, dimp, preferred_element_type=F32)
    p0e_ref[...] += lax.dot_general(onehot, x0, dimp, preferred_element_type=F32)
    cnt_ref[...] += lax.dot_general(onehot, jnp.ones((BLK, D), F32), dimp,
                                    preferred_element_type=F32)
    for r in range(NR):
        h1n_ref[r] = jnp.dot(vx, wr_ref[r], preferred_element_type=F32)
        h1e_ref[r] = jnp.dot(x0, wr_ref[r], preferred_element_type=F32)


def _tc_samplers(x0, macc2, n2g_col, seq_final, w_pn, w_pe, Wg_n, Wg_e, W_R):
    blk = lambda *shape: shape
    return pl.pallas_call(
        _samplers_body,
        grid=(NB,),
        in_specs=[
            pl.BlockSpec((BLK, D), lambda i: (i, 0)),
            pl.BlockSpec((2, BLK, D), lambda i: (0, i, 0)),
            pl.BlockSpec((BLK, 1), lambda i: (i, 0)),
            pl.BlockSpec((B, D), lambda i: (0, 0)),
            pl.BlockSpec((1, D), lambda i: (0, 0)),
            pl.BlockSpec((1, D), lambda i: (0, 0)),
            pl.BlockSpec((D, D), lambda i: (0, 0)),
            pl.BlockSpec((D, D), lambda i: (0, 0)),
            pl.BlockSpec((NR, D, D), lambda i: (0, 0, 0)),
        ],
        out_specs=[
            pl.BlockSpec((BLK, 1), lambda i: (i, 0)),
            pl.BlockSpec((BLK, D), lambda i: (i, 0)),
            pl.BlockSpec((BLK, B), lambda i: (i, 0)),
            pl.BlockSpec((NR, BLK, D), lambda i: (0, i, 0)),
            pl.BlockSpec((NR, BLK, D), lambda i: (0, i, 0)),
            pl.BlockSpec((B, D), lambda i: (0, 0)),
            pl.BlockSpec((B, D), lambda i: (0, 0)),
            pl.BlockSpec((B, D), lambda i: (0, 0)),
        ],
        out_shape=[
            jax.ShapeDtypeStruct((NP, 1), F32),
            jax.ShapeDtypeStruct((NP, D), F32),
            jax.ShapeDtypeStruct((NP, B), F32),
            jax.ShapeDtypeStruct((NR, NP, D), F32),
            jax.ShapeDtypeStruct((NR, NP, D), F32),
            jax.ShapeDtypeStruct((B, D), F32),
            jax.ShapeDtypeStruct((B, D), F32),
            jax.ShapeDtypeStruct((B, D), F32),
        ],
    )(x0, macc2, n2g_col, seq_final, w_pn, w_pe, Wg_n, Wg_e, W_R)


def _layer_body(emit_H, hn_ref, he_ref, an_ref, ae_ref, c_ref, rel_ref,
                n2g_ref, wl_ref, bl_ref, wr_ref, *outs):
    if emit_H:
        hn_o, he_o, H2n_ref, H2e_ref, pn_ref, pe_ref = outs
    else:
        hn_o, he_o, pn_ref, pe_ref = outs
    i = pl.program_id(0)
    relC = jnp.dot(c_ref[0] + c_ref[1], rel_ref[...], preferred_element_type=F32)
    aggn = an_ref[0] + an_ref[1] + relC
    agge = ae_ref[0] + ae_ref[1] + relC
    hn = jnp.maximum(
        jnp.dot(hn_ref[...] + aggn, wl_ref[...], preferred_element_type=F32)
        + bl_ref[...], 0.0)
    he = jnp.maximum(
        jnp.dot(he_ref[...] + agge, wl_ref[...], preferred_element_type=F32)
        + bl_ref[...], 0.0)
    hn_o[...] = hn
    he_o[...] = he
    onehot = (n2g_ref[...] == lax.broadcasted_iota(I32, (1, B), 1)).astype(F32)
    dimp = (((0,), (0,)), ((), ()))

    @pl.when(i == 0)
    def _():
        pn_ref[...] = jnp.zeros((B, D), F32)
        pe_ref[...] = jnp.zeros((B, D), F32)

    pn_ref[...] += lax.dot_general(onehot, hn, dimp, preferred_element_type=F32)
    pe_ref[...] += lax.dot_general(onehot, he, dimp, preferred_element_type=F32)
    if emit_H:
        for r in range(NR):
            H2n_ref[r] = jnp.dot(hn, wr_ref[r], preferred_element_type=F32)
            H2e_ref[r] = jnp.dot(he, wr_ref[r], preferred_element_type=F32)


def _tc_layer(hn, he, accn2, acce2, C2, rel_tab, n2g_col, W_l, b_l, W_R,
              emit_H):
    out_specs = [
        pl.BlockSpec((BLK, D), lambda i: (i, 0)),
        pl.BlockSpec((BLK, D), lambda i: (i, 0)),
    ]
    out_shape = [jax.ShapeDtypeStruct((NP, D), F32),
                 jax.ShapeDtypeStruct((NP, D), F32)]
    if emit_H:
        out_specs += [pl.BlockSpec((NR, BLK, D), lambda i: (0, i, 0))] * 2
        out_shape += [jax.ShapeDtypeStruct((NR, NP, D), F32)] * 2
    out_specs += [pl.BlockSpec((B, D), lambda i: (0, 0))] * 2
    out_shape += [jax.ShapeDtypeStruct((B, D), F32)] * 2
    return pl.pallas_call(
        functools.partial(_layer_body, emit_H),
        grid=(NB,),
        in_specs=[
            pl.BlockSpec((BLK, D), lambda i: (i, 0)),
            pl.BlockSpec((BLK, D), lambda i: (i, 0)),
            pl.BlockSpec((2, BLK, D), lambda i: (0, i, 0)),
            pl.BlockSpec((2, BLK, D), lambda i: (0, i, 0)),
            pl.BlockSpec((2, BLK, NR), lambda i: (0, i, 0)),
            pl.BlockSpec((NR, D), lambda i: (0, 0)),
            pl.BlockSpec((BLK, 1), lambda i: (i, 0)),
            pl.BlockSpec((D, D), lambda i: (0, 0)),
            pl.BlockSpec((1, D), lambda i: (0, 0)),
            pl.BlockSpec((NR, D, D), lambda i: (0, 0, 0)),
        ],
        out_specs=out_specs,
        out_shape=out_shape,
    )(hn, he, accn2, acce2, C2, rel_tab, n2g_col, W_l, b_l, W_R)


def _loss_body(p0n, p1n, p2n, p0e, p1e, p2e, cnt, loss_ref):
    counts = jnp.maximum(cnt[...], 1.0)
    x1 = jnp.concatenate([p0n[...] / counts, p1n[...] / counts,
                          p2n[...] / counts], axis=1)
    x2 = jnp.concatenate([p0e[...] / counts, p1e[...] / counts,
                          p2e[...] / counts], axis=1)
    n1 = jnp.sqrt(jnp.sum(x1 * x1, axis=1, keepdims=True))
    n2 = jnp.sqrt(jnp.sum(x2 * x2, axis=1, keepdims=True))
    dimn = (((1,), (1,)), ((), ()))
    sim = lax.dot_general(x1, x2, dimn, preferred_element_type=F32)
    nn = lax.dot_general(n1, n2, (((1,), (1,)), ((), ())),
                         preferred_element_type=F32)
    ea = jnp.exp(sim / nn / 0.5)
    eye = (lax.broadcasted_iota(I32, (B, B), 0)
           == lax.broadcasted_iota(I32, (B, B), 1)).astype(F32)
    pos = jnp.sum(ea * eye, axis=1, keepdims=True)
    rs = jnp.sum(ea, axis=1, keepdims=True)
    cs = jnp.sum(ea * eye, axis=0, keepdims=True)
    csf = jnp.sum(ea, axis=0, keepdims=True)
    la = -jnp.sum(jnp.log(pos / (rs - pos))) / B
    lb = -jnp.sum(jnp.log(cs / (csf - cs))) / B
    loss_ref[0, 0] = 0.5 * (la + lb)


def _tc_loss(p0n, p1n, p2n, p0e, p1e, p2e, cnt):
    full = pl.BlockSpec((B, D), lambda: (0, 0))
    return pl.pallas_call(
        _loss_body,
        in_specs=[full] * 7,
        out_specs=pl.BlockSpec(memory_space=pltpu.SMEM),
        out_shape=jax.ShapeDtypeStruct((1, 1), F32),
    )(p0n, p1n, p2n, p0e, p1e, p2e, cnt)


_MESH = plsc.VectorSubcoreMesh(core_axis_name="c", subcore_axis_name="s",
                               num_cores=NC, num_subcores=NS)
IPW = NP // NW


def _x0_body(tab, ids, out, idx_v, rows_v, sem):
    wid = lax.axis_index("s") * NC + lax.axis_index("c")
    base = wid * IPW

    def body(j, carry):
        off = base + j * CH
        pltpu.sync_copy(ids.at[pl.ds(off, CH)], idx_v)
        pltpu.async_copy(tab.at[idx_v], rows_v, sem).wait()
        pltpu.sync_copy(rows_v, out.at[pl.ds(off, CH)])
        return carry

    lax.fori_loop(0, IPW // CH, body, 0)


def _sc_x0gather(node_tab, ids_p):
    return pl.kernel(
        _x0_body,
        out_type=jax.ShapeDtypeStruct((NP, D), F32),
        mesh=_MESH,
        scratch_types=[pltpu.VMEM((CH,), I32), pltpu.VMEM((CH, D), F32),
                       pltpu.SemaphoreType.DMA],
    )(node_tab, ids_p)


def _zero_vmem_2d(ref, nrows):
    def body(i, carry):
        r = i // (D // L)
        c = i % (D // L)
        ref[r, pl.ds(c * L, L)] = jnp.zeros((L,), F32)
        return carry
    lax.fori_loop(0, nrows * (D // L), body, 0)


def _zero_vmem_1d(ref, n):
    def body(i, carry):
        ref[pl.ds(i * L, L)] = jnp.zeros((L,), F32)
        return carry
    lax.fori_loop(0, n // L, body, 0)


def _edge_epilogue(acc_sh, acc_out, cid, sid):
    plsc.subcore_barrier()
    for j in range(ROWS_PER_TILE // D):
        off = sid * ROWS_PER_TILE + j * D
        pltpu.sync_copy(acc_sh.at[pl.ds(off, D)], acc_out.at[cid, pl.ds(off, D)])


def _scale_rows(rows_v, get_pvec):
    def body(g, carry):
        pv = get_pvec(g)
        for j in range(L):
            w = jnp.full((L,), pv[j], F32)
            e = g * L + j
            for f in range(D // L):
                rows_v[e, pl.ds(f * L, L)] = rows_v[e, pl.ds(f * L, L)] * w
        return carry
    lax.fori_loop(0, ECH // L, body, 0)


NCHUNK_ROWS = E_PAD // ECH


def _make_edge_pass(nidx, compute_p=False, preload_p=False):
    def body(*refs):
        it = iter(refs)
        tab = next(it); ipack = next(it)
        gef = next(it) if compute_p else None
        p2d_in = next(it) if preload_p else None
        acc_out = next(it)
        pedge_out = next(it) if compute_p else None
        rows = (next(it), next(it))
        ibuf = (next(it), next(it), next(it), next(it))
        if compute_p:
            g1 = (next(it), next(it))
            g2 = (next(it), next(it))
            pbuf = (next(it), next(it))
        p_all = next(it) if preload_p else None
        gsem = (next(it), next(it))
        ssem = (next(it), next(it))
        isem = (next(it), next(it), next(it), next(it))
        psem = (next(it), next(it)) if compute_p else None
        acc_sh = next(it)

        cid = lax.axis_index("c")
        sid = lax.axis_index("s")
        wid = sid * NC + cid
        base = wid * NCH2

        def zr(i, carry):
            r = i // (D // L)
            c = i % (D // L)
            rows[0][r, pl.ds(c * L, L)] = jnp.zeros((L,), F32)
            return carry

        lax.fori_loop(0, ECH * (D // L), zr, 0)
        for j in range(ROWS_PER_TILE // ECH):
            pltpu.sync_copy(
                rows[0], acc_sh.at[pl.ds(sid * ROWS_PER_TILE + j * ECH, ECH)])
        if preload_p:
            pltpu.sync_copy(p2d_in.at[pl.ds(base, NCH2)], p_all)
        plsc.subcore_barrier()

        def issue_gather(k, b2, b3):
            pltpu.async_copy(tab.at[ibuf[b3].at[0]], rows[b2], gsem[b2])
            if compute_p:
                pltpu.async_copy(gef.at[ibuf[b3].at[2]], g1[b2], gsem[b2])
                pltpu.async_copy(gef.at[ibuf[b3].at[3]], g2[b2], gsem[b2])

        def wait_gather(b2, b3):
            pltpu.make_async_copy(tab.at[ibuf[b3].at[0]], rows[b2],
                                  gsem[b2]).wait()
            if compute_p:
                pltpu.make_async_copy(gef.at[ibuf[b3].at[2]], g1[b2],
                                      gsem[b2]).wait()
                pltpu.make_async_copy(gef.at[ibuf[b3].at[3]], g2[b2],
                                      gsem[b2]).wait()

        def wait_scatter(b2, b3):
            pltpu.make_async_copy(rows[b2], acc_sh.at[ibuf[b3].at[1]],
                                  ssem[b2]).wait()

        def do_chunk(k, j):
            b2 = j % 2
            b3 = j % 4

            @pl.when(k >= 1)
            def _():
                wait_scatter(1 - b2, (j + 3) % 4)

            @pl.when(k + 1 < NCH2)
            def _():
                pltpu.make_async_copy(ipack.at[0], ibuf[(j + 1) % 4],
                                      isem[(j + 1) % 4]).wait()
                issue_gather(k + 1, 1 - b2, (j + 1) % 4)

            @pl.when(k + 2 < NCH2)
            def _():
                pltpu.async_copy(ipack.at[base + k + 2], ibuf[(j + 2) % 4],
                                 isem[(j + 2) % 4])

            wait_gather(b2, b3)
            if compute_p:
                @pl.when(k >= 2)
                def _():
                    pltpu.make_async_copy(pbuf[b2], pedge_out.at[0],
                                          psem[b2]).wait()

                def grp(g, carry2):
                    a = g1[b2][pl.ds(g * L, L)]
                    c = g2[b2][pl.ds(g * L, L)]
                    pbuf[b2][pl.ds(g * L, L)] = 1.0 / (1.0 + jnp.exp(-(a + c)))
                    return carry2

                lax.fori_loop(0, ECH // L, grp, 0)
                pltpu.async_copy(pbuf[b2], pedge_out.at[base + k], psem[b2])
                _scale_rows(rows[b2], lambda g: pbuf[b2][pl.ds(g * L, L)])
            elif preload_p:
                _scale_rows(rows[b2], lambda g: p_all[k, pl.ds(g * L, L)])
            pltpu.async_copy(rows[b2], acc_sh.at[ibuf[b3].at[1]], ssem[b2],
                             add=True)

        pltpu.sync_copy(ipack.at[base], ibuf[0])
        pltpu.async_copy(ipack.at[base + 1], ibuf[1], isem[1])
        issue_gather(0, 0, 0)

        def group(g, carry):
            for j in range(4):
                do_chunk(4 * g + j, j)
            return carry

        lax.fori_loop(0, NCH2 // 4, group, 0)
        wait_scatter(1, 3)
        if compute_p:
            pltpu.make_async_copy(pbuf[0], pedge_out.at[0], psem[0]).wait()
            pltpu.make_async_copy(pbuf[1], pedge_out.at[0], psem[1]).wait()
        _edge_epilogue(acc_sh, acc_out, cid, sid)

    out_type = [jax.ShapeDtypeStruct((NC, NP, D), F32)]
    if compute_p:
        out_type.append(jax.ShapeDtypeStruct((NCHUNK_ROWS, ECH), F32))
    scr = [pltpu.VMEM((ECH, D), F32), pltpu.VMEM((ECH, D), F32)]
    scr += [pltpu.VMEM((nidx, ECH), I32)] * 4
    if compute_p:
        scr += [pltpu.VMEM((ECH,), F32)] * 6
    if preload_p:
        scr.append(pltpu.VMEM((NCH2, ECH), F32))
    scr += [pltpu.SemaphoreType.DMA] * 8
    if compute_p:
        scr += [pltpu.SemaphoreType.DMA] * 2
    scr.append(pltpu.VMEM_SHARED((NP, D), F32))

    def run(*args):
        return pl.kernel(
            body,
            out_type=tuple(out_type) if len(out_type) > 1 else out_type[0],
            mesh=_MESH,
            scratch_types=scr,
        )(*args)

    return run


_sc_plain = _make_edge_pass(nidx=2)
_sc_wpass1 = _make_edge_pass(nidx=4, compute_p=True)
_sc_wpass2 = _make_edge_pass(nidx=2, preload_p=True)


def _cpass_body(cidx2, c_out, cidx_all, ones_v, zc_v, csem, c_sh):
    cid = lax.axis_index("c")
    sid = lax.axis_index("s")
    wid = sid * NC + cid
    _zero_vmem_1d(zc_v, NPC // NS // 5)
    for j in range(5):
        pltpu.sync_copy(
            zc_v, c_sh.at[pl.ds(sid * (NPC // NS) + j * (NPC // NS // 5),
                                NPC // NS // 5)])

    def ones_init(i, carry):
        ones_v[pl.ds(i * L, L)] = jnp.ones((L,), F32)
        return carry

    lax.fori_loop(0, ECH // L, ones_init, 0)
    pltpu.sync_copy(cidx2.at[pl.ds(wid * NCH2, NCH2)], cidx_all)
    plsc.subcore_barrier()

    def chunk(k, carry):
        @pl.when(k >= 1)
        def _():
            pltpu.make_async_copy(ones_v, c_sh.at[cidx_all.at[0]],
                                  csem).wait()

        pltpu.async_copy(ones_v, c_sh.at[cidx_all.at[k]], csem, add=True)
        return carry

    lax.fori_loop(0, NCH2, chunk, 0)
    pltpu.make_async_copy(ones_v, c_sh.at[cidx_all.at[0]], csem).wait()
    plsc.subcore_barrier()
    coff = sid * (NPC // NS)
    pltpu.sync_copy(c_sh.at[pl.ds(coff, NPC // NS)],
                    c_out.at[cid, pl.ds(coff, NPC // NS)])


def _sc_cpass(cidx2):
    return pl.kernel(
        _cpass_body,
        out_type=jax.ShapeDtypeStruct((NC, NPC), F32),
        mesh=_MESH,
        scratch_types=[
            pltpu.VMEM((NCH2, ECH), I32),
            pltpu.VMEM((ECH,), F32),
            pltpu.VMEM((NPC // NS // 5,), F32),
            pltpu.SemaphoreType.DMA,
            pltpu.VMEM_SHARED((NPC,), F32),
        ],
    )(cidx2)


def kernel(x_batch, s_batch, s_batch_dim2, batch_mask, node_ids, edge_index,
           edge_type, node2graph, edgebindex, ehrcode_W, node_tab, rel_tab,
           W_R, gru_Wx, gru_Wh, gru_b, att_w, Wg_n, w_pn, Wg_e, w_pe,
           L1_W, L1_b, L2_W, L2_b):
    src = edge_index[0]
    dst = edge_index[1]
    ids_p = jnp.concatenate([node_ids.astype(I32),
                             jnp.full((NP - N,), N, I32)])
    n2g_col = jnp.concatenate([node2graph.astype(I32),
                               jnp.full((NP - N,), B, I32)]).reshape(NP, 1)

    x_tbc = jnp.transpose(x_batch, (1, 0, 2))
    seq_final, alpha = _tc_front(x_tbc, batch_mask, ehrcode_W, gru_Wx,
                                 gru_Wh, gru_b.reshape(1, 3 * D),
                                 att_w.reshape(1, D))

    npad = E_PAD - E
    srcp = jnp.concatenate([src.astype(I32), jnp.full((npad,), N, I32)])
    dstp = jnp.concatenate([dst.astype(I32), jnp.full((npad,), NP - 1, I32)])
    typp = jnp.concatenate([edge_type.astype(I32), jnp.zeros((npad,), I32)])
    ebp = jnp.concatenate([edgebindex.astype(I32), jnp.zeros((npad,), I32)])
    e2 = (E_PAD // D, D)
    srcp = srcp.reshape(e2)
    dstp = dstp.reshape(e2)
    gidx2, cidx2, idx12, idx22 = _tc_edgeidx(
        srcp, dstp, typp.reshape(e2), ebp.reshape(e2))

    ip_m = jnp.stack([srcp, dstp], axis=1)
    ip_g = jnp.stack([gidx2, dstp], axis=1)
    ip_w1 = jnp.stack([gidx2, dstp, idx12, idx22], axis=1)

    x0 = _sc_x0gather(node_tab, ids_p)
    macc2 = _sc_plain(x0, ip_m)
    C2 = _sc_cpass(cidx2).reshape(NC, NP, NR)

    (pn_col, view_x, Ge, H1n, H1e, p0n, p0e, cnt) = _tc_samplers(
        x0, macc2, n2g_col, seq_final, w_pn.reshape(1, D),
        w_pe.reshape(1, D), Wg_n, Wg_e, W_R)

    acc1n = _sc_plain(H1n.reshape(NR * NP, D), ip_g)
    acc1e, pedge2 = _sc_wpass1(H1e.reshape(NR * NP, D), ip_w1,
                               Ge.reshape(NP * B))
    p_edge = pedge2.reshape(E_PAD)[:E]

    h1n, h1e, H2n, H2e, p1n, p1e = _tc_layer(
        view_x, x0, acc1n, acc1e, C2, rel_tab, n2g_col, L1_W,
        L1_b.reshape(1, D), W_R, emit_H=True)

    acc2n = _sc_plain(H2n.reshape(NR * NP, D), ip_g)
    acc2e = _sc_wpass2(H2e.reshape(NR * NP, D), ip_g, pedge2)

    h2n, h2e, p2n, p2e = _tc_layer(
        h1n, h1e, acc2n, acc2e, C2, rel_tab, n2g_col, L2_W,
        L2_b.reshape(1, D), W_R, emit_H=False)

    loss = _tc_loss(p0n, p1n, p2n, p0e, p1e, p2e, cnt)[0, 0]

    p_node = pn_col.reshape(NP)[:N]
    return (loss, p_node, p_node, p_edge, p_edge, seq_final, alpha)


# speedup vs baseline: 1.0409x; 1.0409x over previous
import functools

import jax
import jax.numpy as jnp
from jax import lax
from jax.experimental import pallas as pl
from jax.experimental.pallas import tpu as pltpu
import jax.experimental.pallas.tpu_sc as plsc

B = 16; T = 20; CODE = 2000; D = 128; N = 10000; E = 320000; R = 16
E_PAD = 327680
ECH = 128
NP = 10240
NB = 10
BLK = NP // NB
NR = R + 1
NPC = NP * NR
NC, NS, L = 2, 16, 16
NW = NC * NS
CH = 80
NCH2 = E_PAD // NW // ECH
ROWS_PER_TILE = NP // NS
F32 = jnp.float32
I32 = jnp.int32


def _sigmoid(x):
    return 1.0 / (1.0 + jnp.exp(-x))


def _front_body(x_ref, mask_ref, ehr_ref, wx_ref, wh_ref, b_ref, aw_ref,
                sf_ref, alpha_ref, hs_scr, h_scr, sc_scr):
    t = pl.program_id(0)

    @pl.when(t == 0)
    def _():
        h_scr[...] = jnp.zeros((B, D), F32)
        sc_scr[...] = jnp.zeros((B, D), F32)

    h = h_scr[...]
    xt = x_ref[0]
    e = jnp.dot(xt, ehr_ref[...], preferred_element_type=F32)
    gx = jnp.dot(e, wx_ref[...], preferred_element_type=F32) + b_ref[...]
    gh = jnp.dot(h, wh_ref[...], preferred_element_type=F32)
    z = _sigmoid(gx[:, :D] + gh[:, :D])
    r = _sigmoid(gx[:, D:2 * D] + gh[:, D:2 * D])
    n = jnp.tanh(gx[:, 2 * D:] + r * gh[:, 2 * D:])
    h = (1.0 - z) * n + z * h
    h_scr[...] = h
    hs_scr[t] = h
    sval = jnp.sum(jnp.tanh(h) * aw_ref[...], axis=1, keepdims=True)
    lane = lax.broadcasted_iota(I32, (B, D), 1)
    sc_scr[...] += jnp.where(lane == t, sval, 0.0)

    @pl.when(t == T - 1)
    def _():
        s = sc_scr[:, :T] + (mask_ref[...] - 1.0) * 1e9
        smax = jnp.max(s, axis=1, keepdims=True)
        ex = jnp.exp(s - smax)
        alpha = ex / jnp.sum(ex, axis=1, keepdims=True)
        alpha_ref[...] = alpha
        lane20 = lax.broadcasted_iota(I32, (B, T), 1)

        def acc(i, carry):
            a_i = jnp.sum(jnp.where(lane20 == i, alpha, 0.0), axis=1,
                          keepdims=True)
            return carry + a_i * hs_scr[i]

        sf_ref[...] = lax.fori_loop(0, T, acc, jnp.zeros((B, D), F32))


def _tc_front(x_tbc, batch_mask, ehrcode_W, gru_Wx, gru_Wh, gru_b, att_w):
    full = lambda shape: pl.BlockSpec(shape, lambda t: (0,) * len(shape))
    return pl.pallas_call(
        _front_body,
        grid=(T,),
        in_specs=[
            pl.BlockSpec((1, B, CODE), lambda t: (t, 0, 0)),
            full((B, T)), full((CODE, D)), full((D, 3 * D)),
            full((D, 3 * D)), full((1, 3 * D)), full((1, D)),
        ],
        out_specs=[full((B, D)), full((B, T))],
        out_shape=[jax.ShapeDtypeStruct((B, D), F32),
                   jax.ShapeDtypeStruct((B, T), F32)],
        scratch_shapes=[pltpu.VMEM((T, B, D), F32), pltpu.VMEM((B, D), F32),
                        pltpu.VMEM((B, D), F32)],
    )(x_tbc, batch_mask, ehrcode_W, gru_Wx, gru_Wh, gru_b, att_w)


def _edgeidx_body(src_ref, dst_ref, typ_ref, eb_ref, gidx_ref, cidx_ref,
                  i1_ref, i2_ref):
    s = src_ref[...]
    d = dst_ref[...]
    t = typ_ref[...]
    eb = eb_ref[...]
    gidx_ref[...] = t * NP + s
    cidx_ref[...] = d * NR + t
    i1_ref[...] = s * B + eb
    i2_ref[...] = d * B + eb


def _tc_edgeidx(src2d, dst2d, typ2d, eb2d):
    sh = src2d.shape
    full = pl.BlockSpec(sh, lambda: (0, 0))
    return pl.pallas_call(
        _edgeidx_body,
        in_specs=[full] * 4,
        out_specs=[full] * 4,
        out_shape=[jax.ShapeDtypeStruct(sh, I32)] * 4,
    )(src2d, dst2d, typ2d, eb2d)


def _samplers_body(x0_ref, macc_ref, n2g_ref, sf_ref, wpn_ref, wpe_ref,
                   wgn_ref, wge_ref, wr_ref,
                   pn_ref, vx_ref, ge_ref, h1n_ref, h1e_ref,
                   p0n_ref, p0e_ref, cnt_ref):
    i = pl.program_id(0)
    x0 = x0_ref[...]
    xm = x0 + macc_ref[0] + macc_ref[1]
    hg = jnp.maximum(jnp.dot(xm, wgn_ref[...], preferred_element_type=F32), 0.0)
    hge = jnp.maximum(jnp.dot(xm, wge_ref[...], preferred_element_type=F32), 0.0)
    qn = sf_ref[...] + wpn_ref[...]
    qe = sf_ref[...] + wpe_ref[...]
    dimn = (((1,), (1,)), ((), ()))
    Gn = lax.dot_general(hg, qn, dimn, preferred_element_type=F32)
    Ge = lax.dot_general(hge, qe, dimn, preferred_element_type=F32)
    ge_ref[...] = Ge
    onehot = (n2g_ref[...] == lax.broadcasted_iota(I32, (1, B), 1)).astype(F32)
    pick = jnp.sum(Gn * onehot, axis=1, keepdims=True)
    p = _sigmoid(pick)
    pn_ref[...] = p
    vx = x0 * p
    vx_ref[...] = vx
    dimp = (((0,), (0,)), ((), ()))

    @pl.when(i == 0)
    def _():
        p0n_ref[...] = jnp.zeros((B, D), F32)
        p0e_ref[...] = jnp.zeros((B, D), F32)
        cnt_ref[...] = jnp.zeros((B, D), F32)

    p0n_ref[...] += lax.dot_general(onehot, vx, dimp, preferred_element_type=F32)
    p0e_ref[...] += lax.dot_general(onehot, x0, dimp, preferred_element_type=F32)
    cnt_ref[...] += lax.dot_general(onehot, jnp.ones((BLK, D), F32), dimp,
                                    preferred_element_type=F32)
    for r in range(NR):
        h1n_ref[r] = jnp.dot(vx, wr_ref[r], preferred_element_type=F32)
        h1e_ref[r] = jnp.dot(x0, wr_ref[r], preferred_element_type=F32)


def _tc_samplers(x0, macc2, n2g_col, seq_final, w_pn, w_pe, Wg_n, Wg_e, W_R):
    blk = lambda *shape: shape
    return pl.pallas_call(
        _samplers_body,
        grid=(NB,),
        in_specs=[
            pl.BlockSpec((BLK, D), lambda i: (i, 0)),
            pl.BlockSpec((2, BLK, D), lambda i: (0, i, 0)),
            pl.BlockSpec((BLK, 1), lambda i: (i, 0)),
            pl.BlockSpec((B, D), lambda i: (0, 0)),
            pl.BlockSpec((1, D), lambda i: (0, 0)),
            pl.BlockSpec((1, D), lambda i: (0, 0)),
            pl.BlockSpec((D, D), lambda i: (0, 0)),
            pl.BlockSpec((D, D), lambda i: (0, 0)),
            pl.BlockSpec((NR, D, D), lambda i: (0, 0, 0)),
        ],
        out_specs=[
            pl.BlockSpec((BLK, 1), lambda i: (i, 0)),
            pl.BlockSpec((BLK, D), lambda i: (i, 0)),
            pl.BlockSpec((BLK, B), lambda i: (i, 0)),
            pl.BlockSpec((NR, BLK, D), lambda i: (0, i, 0)),
            pl.BlockSpec((NR, BLK, D), lambda i: (0, i, 0)),
            pl.BlockSpec((B, D), lambda i: (0, 0)),
            pl.BlockSpec((B, D), lambda i: (0, 0)),
            pl.BlockSpec((B, D), lambda i: (0, 0)),
        ],
        out_shape=[
            jax.ShapeDtypeStruct((NP, 1), F32),
            jax.ShapeDtypeStruct((NP, D), F32),
            jax.ShapeDtypeStruct((NP, B), F32),
            jax.ShapeDtypeStruct((NR, NP, D), F32),
            jax.ShapeDtypeStruct((NR, NP, D), F32),
            jax.ShapeDtypeStruct((B, D), F32),
            jax.ShapeDtypeStruct((B, D), F32),
            jax.ShapeDtypeStruct((B, D), F32),
        ],
    )(x0, macc2, n2g_col, seq_final, w_pn, w_pe, Wg_n, Wg_e, W_R)


def _layer_body(emit_H, hn_ref, he_ref, an_ref, ae_ref, c_ref, rel_ref,
                n2g_ref, wl_ref, bl_ref, wr_ref, *outs):
    if emit_H:
        hn_o, he_o, H2n_ref, H2e_ref, pn_ref, pe_ref = outs
    else:
        hn_o, he_o, pn_ref, pe_ref = outs
    i = pl.program_id(0)
    relC = jnp.dot(c_ref[0] + c_ref[1], rel_ref[...], preferred_element_type=F32)
    aggn = an_ref[0] + an_ref[1] + relC
    agge = ae_ref[0] + ae_ref[1] + relC
    hn = jnp.maximum(
        jnp.dot(hn_ref[...] + aggn, wl_ref[...], preferred_element_type=F32)
        + bl_ref[...], 0.0)
    he = jnp.maximum(
        jnp.dot(he_ref[...] + agge, wl_ref[...], preferred_element_type=F32)
        + bl_ref[...], 0.0)
    hn_o[...] = hn
    he_o[...] = he
    onehot = (n2g_ref[...] == lax.broadcasted_iota(I32, (1, B), 1)).astype(F32)
    dimp = (((0,), (0,)), ((), ()))

    @pl.when(i == 0)
    def _():
        pn_ref[...] = jnp.zeros((B, D), F32)
        pe_ref[...] = jnp.zeros((B, D), F32)

    pn_ref[...] += lax.dot_general(onehot, hn, dimp, preferred_element_type=F32)
    pe_ref[...] += lax.dot_general(onehot, he, dimp, preferred_element_type=F32)
    if emit_H:
        for r in range(NR):
            H2n_ref[r] = jnp.dot(hn, wr_ref[r], preferred_element_type=F32)
            H2e_ref[r] = jnp.dot(he, wr_ref[r], preferred_element_type=F32)


def _tc_layer(hn, he, accn2, acce2, C2, rel_tab, n2g_col, W_l, b_l, W_R,
              emit_H):
    out_specs = [
        pl.BlockSpec((BLK, D), lambda i: (i, 0)),
        pl.BlockSpec((BLK, D), lambda i: (i, 0)),
    ]
    out_shape = [jax.ShapeDtypeStruct((NP, D), F32),
                 jax.ShapeDtypeStruct((NP, D), F32)]
    if emit_H:
        out_specs += [pl.BlockSpec((NR, BLK, D), lambda i: (0, i, 0))] * 2
        out_shape += [jax.ShapeDtypeStruct((NR, NP, D), F32)] * 2
    out_specs += [pl.BlockSpec((B, D), lambda i: (0, 0))] * 2
    out_shape += [jax.ShapeDtypeStruct((B, D), F32)] * 2
    return pl.pallas_call(
        functools.partial(_layer_body, emit_H),
        grid=(NB,),
        in_specs=[
            pl.BlockSpec((BLK, D), lambda i: (i, 0)),
            pl.BlockSpec((BLK, D), lambda i: (i, 0)),
            pl.BlockSpec((2, BLK, D), lambda i: (0, i, 0)),
            pl.BlockSpec((2, BLK, D), lambda i: (0, i, 0)),
            pl.BlockSpec((2, BLK, NR), lambda i: (0, i, 0)),
            pl.BlockSpec((NR, D), lambda i: (0, 0)),
            pl.BlockSpec((BLK, 1), lambda i: (i, 0)),
            pl.BlockSpec((D, D), lambda i: (0, 0)),
            pl.BlockSpec((1, D), lambda i: (0, 0)),
            pl.BlockSpec((NR, D, D), lambda i: (0, 0, 0)),
        ],
        out_specs=out_specs,
        out_shape=out_shape,
    )(hn, he, accn2, acce2, C2, rel_tab, n2g_col, W_l, b_l, W_R)


def _loss_body(p0n, p1n, p2n, p0e, p1e, p2e, cnt, loss_ref):
    counts = jnp.maximum(cnt[...], 1.0)
    x1 = jnp.concatenate([p0n[...] / counts, p1n[...] / counts,
                          p2n[...] / counts], axis=1)
    x2 = jnp.concatenate([p0e[...] / counts, p1e[...] / counts,
                          p2e[...] / counts], axis=1)
    n1 = jnp.sqrt(jnp.sum(x1 * x1, axis=1, keepdims=True))
    n2 = jnp.sqrt(jnp.sum(x2 * x2, axis=1, keepdims=True))
    dimn = (((1,), (1,)), ((), ()))
    sim = lax.dot_general(x1, x2, dimn, preferred_element_type=F32)
    nn = lax.dot_general(n1, n2, (((1,), (1,)), ((), ())),
                         preferred_element_type=F32)
    ea = jnp.exp(sim / nn / 0.5)
    eye = (lax.broadcasted_iota(I32, (B, B), 0)
           == lax.broadcasted_iota(I32, (B, B), 1)).astype(F32)
    pos = jnp.sum(ea * eye, axis=1, keepdims=True)
    rs = jnp.sum(ea, axis=1, keepdims=True)
    cs = jnp.sum(ea * eye, axis=0, keepdims=True)
    csf = jnp.sum(ea, axis=0, keepdims=True)
    la = -jnp.sum(jnp.log(pos / (rs - pos))) / B
    lb = -jnp.sum(jnp.log(cs / (csf - cs))) / B
    loss_ref[0, 0] = 0.5 * (la + lb)


def _tc_loss(p0n, p1n, p2n, p0e, p1e, p2e, cnt):
    full = pl.BlockSpec((B, D), lambda: (0, 0))
    return pl.pallas_call(
        _loss_body,
        in_specs=[full] * 7,
        out_specs=pl.BlockSpec(memory_space=pltpu.SMEM),
        out_shape=jax.ShapeDtypeStruct((1, 1), F32),
    )(p0n, p1n, p2n, p0e, p1e, p2e, cnt)


_MESH = plsc.VectorSubcoreMesh(core_axis_name="c", subcore_axis_name="s",
                               num_cores=NC, num_subcores=NS)
IPW = NP // NW


def _x0_body(tab, ids, out, idx_v, rows_v, sem):
    wid = lax.axis_index("s") * NC + lax.axis_index("c")
    base = wid * IPW

    def body(j, carry):
        off = base + j * CH
        pltpu.sync_copy(ids.at[pl.ds(off, CH)], idx_v)
        pltpu.async_copy(tab.at[idx_v], rows_v, sem).wait()
        pltpu.sync_copy(rows_v, out.at[pl.ds(off, CH)])
        return carry

    lax.fori_loop(0, IPW // CH, body, 0)


def _sc_x0gather(node_tab, ids_p):
    return pl.kernel(
        _x0_body,
        out_type=jax.ShapeDtypeStruct((NP, D), F32),
        mesh=_MESH,
        scratch_types=[pltpu.VMEM((CH,), I32), pltpu.VMEM((CH, D), F32),
                       pltpu.SemaphoreType.DMA],
    )(node_tab, ids_p)


def _zero_vmem_2d(ref, nrows):
    def body(i, carry):
        r = i // (D // L)
        c = i % (D // L)
        ref[r, pl.ds(c * L, L)] = jnp.zeros((L,), F32)
        return carry
    lax.fori_loop(0, nrows * (D // L), body, 0)


def _zero_vmem_1d(ref, n):
    def body(i, carry):
        ref[pl.ds(i * L, L)] = jnp.zeros((L,), F32)
        return carry
    lax.fori_loop(0, n // L, body, 0)


def _edge_epilogue(acc_sh, acc_out, cid, sid):
    plsc.subcore_barrier()
    for j in range(ROWS_PER_TILE // D):
        off = sid * ROWS_PER_TILE + j * D
        pltpu.sync_copy(acc_sh.at[pl.ds(off, D)], acc_out.at[cid, pl.ds(off, D)])


def _scale_rows(rows_v, get_pvec):
    def body(g, carry):
        pv = get_pvec(g)
        for j in range(L):
            w = jnp.full((L,), pv[j], F32)
            e = g * L + j
            for f in range(D // L):
                rows_v[e, pl.ds(f * L, L)] = rows_v[e, pl.ds(f * L, L)] * w
        return carry
    lax.fori_loop(0, ECH // L, body, 0)


NCHUNK_ROWS = E_PAD // ECH
KA = 132
KB = 28


def _make_edge_pass(nidx, compute_p=False, preload_p=False):
    def body(*refs):
        it = iter(refs)
        tab = next(it); ipack = next(it)
        gef = next(it) if compute_p else None
        acc_out = next(it)
        pedge_out = next(it) if compute_p else None
        rows = (next(it), next(it))
        ibuf = (next(it), next(it), next(it), next(it))
        if compute_p:
            g1 = (next(it), next(it))
            g2 = (next(it), next(it))
            pbuf = (next(it), next(it))
        gsem = (next(it), next(it))
        ssem = (next(it), next(it))
        isem = (next(it), next(it), next(it), next(it))
        psem = (next(it), next(it)) if compute_p else None
        acc_sh = next(it)

        cid = lax.axis_index("c")
        sid = lax.axis_index("s")

        def zr(i, carry):
            r = i // (D // L)
            c = i % (D // L)
            rows[0][r, pl.ds(c * L, L)] = jnp.zeros((L,), F32)
            return carry

        lax.fori_loop(0, ECH * (D // L), zr, 0)
        for j in range(ROWS_PER_TILE // ECH):
            pltpu.sync_copy(
                rows[0], acc_sh.at[pl.ds(sid * ROWS_PER_TILE + j * ECH, ECH)])
        plsc.subcore_barrier()

        def issue_gather(b2, b3):
            pltpu.async_copy(tab.at[ibuf[b3].at[0]], rows[b2], gsem[b2])
            if compute_p:
                pltpu.async_copy(gef.at[ibuf[b3].at[2]], g1[b2], gsem[b2])
                pltpu.async_copy(gef.at[ibuf[b3].at[3]], g2[b2], gsem[b2])

        def wait_gather(b2, b3):
            pltpu.make_async_copy(tab.at[ibuf[b3].at[0]], rows[b2],
                                  gsem[b2]).wait()
            if compute_p:
                pltpu.make_async_copy(gef.at[ibuf[b3].at[2]], g1[b2],
                                      gsem[b2]).wait()
                pltpu.make_async_copy(gef.at[ibuf[b3].at[3]], g2[b2],
                                      gsem[b2]).wait()

        def wait_scatter(b2, b3):
            pltpu.make_async_copy(rows[b2], acc_sh.at[ibuf[b3].at[1]],
                                  ssem[b2]).wait()

        def pipeline(cbase, nch):
            def do_chunk(k, j):
                b2 = j % 2
                b3 = j % 4

                @pl.when(k >= 1)
                def _():
                    wait_scatter(1 - b2, (j + 3) % 4)

                @pl.when(k + 1 < nch)
                def _():
                    pltpu.make_async_copy(ipack.at[0], ibuf[(j + 1) % 4],
                                          isem[(j + 1) % 4]).wait()
                    issue_gather(1 - b2, (j + 1) % 4)

                @pl.when(k + 2 < nch)
                def _():
                    pltpu.async_copy(ipack.at[cbase + k + 2],
                                     ibuf[(j + 2) % 4], isem[(j + 2) % 4])

                wait_gather(b2, b3)
                if compute_p:
                    @pl.when(k >= 2)
                    def _():
                        pltpu.make_async_copy(pbuf[b2], pedge_out.at[0],
                                              psem[b2]).wait()

                    def grp(g, carry2):
                        a = g1[b2][pl.ds(g * L, L)]
                        c = g2[b2][pl.ds(g * L, L)]
                        pbuf[b2][pl.ds(g * L, L)] = 1.0 / (
                            1.0 + jnp.exp(-(a + c)))
                        return carry2

                    lax.fori_loop(0, ECH // L, grp, 0)
                    pltpu.async_copy(pbuf[b2], pedge_out.at[cbase + k],
                                     psem[b2])
                    _scale_rows(rows[b2], lambda g: pbuf[b2][pl.ds(g * L, L)])
                elif preload_p:
                    _scale_rows(
                        rows[b2],
                        lambda g: lax.bitcast_convert_type(
                            ibuf[b3][2, pl.ds(g * L, L)], F32))
                pltpu.async_copy(rows[b2], acc_sh.at[ibuf[b3].at[1]],
                                 ssem[b2], add=True)

            pltpu.sync_copy(ipack.at[cbase], ibuf[0])
            pltpu.async_copy(ipack.at[cbase + 1], ibuf[1], isem[1])
            issue_gather(0, 0)

            def group(g, carry):
                for j in range(4):
                    do_chunk(4 * g + j, j)
                return carry

            lax.fori_loop(0, nch // 4, group, 0)
            wait_scatter(1, 3)
            if compute_p:
                pltpu.make_async_copy(pbuf[0], pedge_out.at[0], psem[0]).wait()
                pltpu.make_async_copy(pbuf[1], pedge_out.at[0], psem[1]).wait()

        @pl.when(cid == 0)
        def _():
            pipeline(sid * KA, KA)

        @pl.when(cid == 1)
        def _():
            pipeline(NS * KA + sid * KB, KB)

        _edge_epilogue(acc_sh, acc_out, cid, sid)

    out_type = [jax.ShapeDtypeStruct((NC, NP, D), F32)]
    if compute_p:
        out_type.append(jax.ShapeDtypeStruct((NCHUNK_ROWS, ECH), F32))
    scr = [pltpu.VMEM((ECH, D), F32), pltpu.VMEM((ECH, D), F32)]
    scr += [pltpu.VMEM((nidx, ECH), I32)] * 4
    if compute_p:
        scr += [pltpu.VMEM((ECH,), F32)] * 6
    scr += [pltpu.SemaphoreType.DMA] * 8
    if compute_p:
        scr += [pltpu.SemaphoreType.DMA] * 2
    scr.append(pltpu.VMEM_SHARED((NP, D), F32))

    def run(*args):
        return pl.kernel(
            body,
            out_type=tuple(out_type) if len(out_type) > 1 else out_type[0],
            mesh=_MESH,
            scratch_types=scr,
        )(*args)

    return run


_sc_plain = _make_edge_pass(nidx=2)
_sc_wpass1 = _make_edge_pass(nidx=4, compute_p=True)
_sc_wpass2 = _make_edge_pass(nidx=3, preload_p=True)


def _cpass_body(cidx2, c_out, cidx_all, ones_v, zc_v, csem, c_sh):
    cid = lax.axis_index("c")
    sid = lax.axis_index("s")
    wid = sid * NC + cid
    _zero_vmem_1d(zc_v, NPC // NS // 5)
    for j in range(5):
        pltpu.sync_copy(
            zc_v, c_sh.at[pl.ds(sid * (NPC // NS) + j * (NPC // NS // 5),
                                NPC // NS // 5)])

    def ones_init(i, carry):
        ones_v[pl.ds(i * L, L)] = jnp.ones((L,), F32)
        return carry

    lax.fori_loop(0, ECH // L, ones_init, 0)
    pltpu.sync_copy(cidx2.at[pl.ds(wid * NCH2, NCH2)], cidx_all)
    plsc.subcore_barrier()

    def chunk(k, carry):
        @pl.when(k >= 1)
        def _():
            pltpu.make_async_copy(ones_v, c_sh.at[cidx_all.at[0]],
                                  csem).wait()

        pltpu.async_copy(ones_v, c_sh.at[cidx_all.at[k]], csem, add=True)
        return carry

    lax.fori_loop(0, NCH2, chunk, 0)
    pltpu.make_async_copy(ones_v, c_sh.at[cidx_all.at[0]], csem).wait()
    plsc.subcore_barrier()
    coff = sid * (NPC // NS)
    pltpu.sync_copy(c_sh.at[pl.ds(coff, NPC // NS)],
                    c_out.at[cid, pl.ds(coff, NPC // NS)])


def _sc_cpass(cidx2):
    return pl.kernel(
        _cpass_body,
        out_type=jax.ShapeDtypeStruct((NC, NPC), F32),
        mesh=_MESH,
        scratch_types=[
            pltpu.VMEM((NCH2, ECH), I32),
            pltpu.VMEM((ECH,), F32),
            pltpu.VMEM((NPC // NS // 5,), F32),
            pltpu.SemaphoreType.DMA,
            pltpu.VMEM_SHARED((NPC,), F32),
        ],
    )(cidx2)


def kernel(x_batch, s_batch, s_batch_dim2, batch_mask, node_ids, edge_index,
           edge_type, node2graph, edgebindex, ehrcode_W, node_tab, rel_tab,
           W_R, gru_Wx, gru_Wh, gru_b, att_w, Wg_n, w_pn, Wg_e, w_pe,
           L1_W, L1_b, L2_W, L2_b):
    src = edge_index[0]
    dst = edge_index[1]
    ids_p = jnp.concatenate([node_ids.astype(I32),
                             jnp.full((NP - N,), N, I32)])
    n2g_col = jnp.concatenate([node2graph.astype(I32),
                               jnp.full((NP - N,), B, I32)]).reshape(NP, 1)

    x_tbc = jnp.transpose(x_batch, (1, 0, 2))
    seq_final, alpha = _tc_front(x_tbc, batch_mask, ehrcode_W, gru_Wx,
                                 gru_Wh, gru_b.reshape(1, 3 * D),
                                 att_w.reshape(1, D))

    npad = E_PAD - E
    srcp = jnp.concatenate([src.astype(I32), jnp.full((npad,), N, I32)])
    dstp = jnp.concatenate([dst.astype(I32), jnp.full((npad,), NP - 1, I32)])
    typp = jnp.concatenate([edge_type.astype(I32), jnp.zeros((npad,), I32)])
    ebp = jnp.concatenate([edgebindex.astype(I32), jnp.zeros((npad,), I32)])
    e2 = (E_PAD // D, D)
    srcp = srcp.reshape(e2)
    dstp = dstp.reshape(e2)
    gidx2, cidx2, idx12, idx22 = _tc_edgeidx(
        srcp, dstp, typp.reshape(e2), ebp.reshape(e2))

    ip_m = jnp.stack([srcp, dstp], axis=1)
    ip_g = jnp.stack([gidx2, dstp], axis=1)
    ip_w1 = jnp.stack([gidx2, dstp, idx12, idx22], axis=1)

    x0 = _sc_x0gather(node_tab, ids_p)
    macc2 = _sc_plain(x0, ip_m)
    C2 = _sc_cpass(cidx2).reshape(NC, NP, NR)

    (pn_col, view_x, Ge, H1n, H1e, p0n, p0e, cnt) = _tc_samplers(
        x0, macc2, n2g_col, seq_final, w_pn.reshape(1, D),
        w_pe.reshape(1, D), Wg_n, Wg_e, W_R)

    acc1n = _sc_plain(H1n.reshape(NR * NP, D), ip_g)
    acc1e, pedge2 = _sc_wpass1(H1e.reshape(NR * NP, D), ip_w1,
                               Ge.reshape(NP * B))
    p_edge = pedge2.reshape(E_PAD)[:E]

    h1n, h1e, H2n, H2e, p1n, p1e = _tc_layer(
        view_x, x0, acc1n, acc1e, C2, rel_tab, n2g_col, L1_W,
        L1_b.reshape(1, D), W_R, emit_H=True)

    acc2n = _sc_plain(H2n.reshape(NR * NP, D), ip_g)
    ip_w2 = jnp.stack(
        [gidx2, dstp, lax.bitcast_convert_type(pedge2, I32)], axis=1)
    acc2e = _sc_wpass2(H2e.reshape(NR * NP, D), ip_w2)

    h2n, h2e, p2n, p2e = _tc_layer(
        h1n, h1e, acc2n, acc2e, C2, rel_tab, n2g_col, L2_W,
        L2_b.reshape(1, D), W_R, emit_H=False)

    loss = _tc_loss(p0n, p1n, p2n, p0e, p1e, p2e, cnt)[0, 0]

    p_node = pn_col.reshape(NP)[:N]
    return (loss, p_node, p_node, p_edge, p_edge, seq_final, alpha)


# speedup vs baseline: 1.1953x; 1.1483x over previous
import functools

import jax
import jax.numpy as jnp
from jax import lax
from jax.experimental import pallas as pl
from jax.experimental.pallas import tpu as pltpu
import jax.experimental.pallas.tpu_sc as plsc

B = 16; T = 20; CODE = 2000; D = 128; N = 10000; E = 320000; R = 16
E_PAD = 327680
ECH = 128
NP = 10240
NB = 10
BLK = NP // NB
NR = R + 1
NPC = NP * NR
NC, NS, L = 2, 16, 16
NW = NC * NS
CH = 80
NCH2 = E_PAD // NW // ECH
ROWS_PER_TILE = NP // NS
F32 = jnp.float32
I32 = jnp.int32


def _sigmoid(x):
    return 1.0 / (1.0 + jnp.exp(-x))


def _front_body(x_ref, mask_ref, ehr_ref, wx_ref, wh_ref, b_ref, aw_ref,
                sf_ref, alpha_ref, hs_scr, h_scr, sc_scr):
    t = pl.program_id(0)

    @pl.when(t == 0)
    def _():
        h_scr[...] = jnp.zeros((B, D), F32)
        sc_scr[...] = jnp.zeros((B, D), F32)

    h = h_scr[...]
    xt = x_ref[0]
    e = jnp.dot(xt, ehr_ref[...], preferred_element_type=F32)
    gx = jnp.dot(e, wx_ref[...], preferred_element_type=F32) + b_ref[...]
    gh = jnp.dot(h, wh_ref[...], preferred_element_type=F32)
    z = _sigmoid(gx[:, :D] + gh[:, :D])
    r = _sigmoid(gx[:, D:2 * D] + gh[:, D:2 * D])
    n = jnp.tanh(gx[:, 2 * D:] + r * gh[:, 2 * D:])
    h = (1.0 - z) * n + z * h
    h_scr[...] = h
    hs_scr[t] = h
    sval = jnp.sum(jnp.tanh(h) * aw_ref[...], axis=1, keepdims=True)
    lane = lax.broadcasted_iota(I32, (B, D), 1)
    sc_scr[...] += jnp.where(lane == t, sval, 0.0)

    @pl.when(t == T - 1)
    def _():
        s = sc_scr[:, :T] + (mask_ref[...] - 1.0) * 1e9
        smax = jnp.max(s, axis=1, keepdims=True)
        ex = jnp.exp(s - smax)
        alpha = ex / jnp.sum(ex, axis=1, keepdims=True)
        alpha_ref[...] = alpha
        lane20 = lax.broadcasted_iota(I32, (B, T), 1)

        def acc(i, carry):
            a_i = jnp.sum(jnp.where(lane20 == i, alpha, 0.0), axis=1,
                          keepdims=True)
            return carry + a_i * hs_scr[i]

        sf_ref[...] = lax.fori_loop(0, T, acc, jnp.zeros((B, D), F32))


def _tc_front(x_tbc, batch_mask, ehrcode_W, gru_Wx, gru_Wh, gru_b, att_w):
    full = lambda shape: pl.BlockSpec(shape, lambda t: (0,) * len(shape))
    return pl.pallas_call(
        _front_body,
        grid=(T,),
        in_specs=[
            pl.BlockSpec((1, B, CODE), lambda t: (t, 0, 0)),
            full((B, T)), full((CODE, D)), full((D, 3 * D)),
            full((D, 3 * D)), full((1, 3 * D)), full((1, D)),
        ],
        out_specs=[full((B, D)), full((B, T))],
        out_shape=[jax.ShapeDtypeStruct((B, D), F32),
                   jax.ShapeDtypeStruct((B, T), F32)],
        scratch_shapes=[pltpu.VMEM((T, B, D), F32), pltpu.VMEM((B, D), F32),
                        pltpu.VMEM((B, D), F32)],
    )(x_tbc, batch_mask, ehrcode_W, gru_Wx, gru_Wh, gru_b, att_w)


def _edgeidx_body(src_ref, dst_ref, typ_ref, eb_ref, gidx_ref, cidx_ref,
                  i1_ref, i2_ref):
    s = src_ref[...]
    d = dst_ref[...]
    t = typ_ref[...]
    eb = eb_ref[...]
    gidx_ref[...] = t * NP + s
    cidx_ref[...] = d * NR + t
    i1_ref[...] = s * B + eb
    i2_ref[...] = d * B + eb


def _tc_edgeidx(src2d, dst2d, typ2d, eb2d):
    sh = src2d.shape
    full = pl.BlockSpec(sh, lambda: (0, 0))
    return pl.pallas_call(
        _edgeidx_body,
        in_specs=[full] * 4,
        out_specs=[full] * 4,
        out_shape=[jax.ShapeDtypeStruct(sh, I32)] * 4,
    )(src2d, dst2d, typ2d, eb2d)


def _samplers_body(x0_ref, macc_ref, n2g_ref, sf_ref, wpn_ref, wpe_ref,
                   wgn_ref, wge_ref, wr_ref,
                   pn_ref, vx_ref, ge_ref, h1n_ref, h1e_ref,
                   p0n_ref, p0e_ref, cnt_ref):
    i = pl.program_id(0)
    x0 = x0_ref[...]
    xm = x0 + macc_ref[0] + macc_ref[1]
    hg = jnp.maximum(jnp.dot(xm, wgn_ref[...], preferred_element_type=F32), 0.0)
    hge = jnp.maximum(jnp.dot(xm, wge_ref[...], preferred_element_type=F32), 0.0)
    qn = sf_ref[...] + wpn_ref[...]
    qe = sf_ref[...] + wpe_ref[...]
    dimn = (((1,), (1,)), ((), ()))
    Gn = lax.dot_general(hg, qn, dimn, preferred_element_type=F32)
    Ge = lax.dot_general(hge, qe, dimn, preferred_element_type=F32)
    ge_ref[...] = Ge
    onehot = (n2g_ref[...] == lax.broadcasted_iota(I32, (1, B), 1)).astype(F32)
    pick = jnp.sum(Gn * onehot, axis=1, keepdims=True)
    p = _sigmoid(pick)
    pn_ref[...] = p
    vx = x0 * p
    vx_ref[...] = vx
    dimp = (((0,), (0,)), ((), ()))

    @pl.when(i == 0)
    def _():
        p0n_ref[...] = jnp.zeros((B, D), F32)
        p0e_ref[...] = jnp.zeros((B, D), F32)
        cnt_ref[...] = jnp.zeros((B, D), F32)

    p0n_ref[...] += lax.dot_general(onehot, vx, dimp, preferred_element_type=F32)
    p0e_ref[...] += lax.dot_general(onehot, x0, dimp, preferred_element_type=F32)
    cnt_ref[...] += lax.dot_general(onehot, jnp.ones((BLK, D), F32), dimp,
                                    preferred_element_type=F32)
    for r in range(NR):
        h1n_ref[r] = jnp.dot(vx, wr_ref[r], preferred_element_type=F32)
        h1e_ref[r] = jnp.dot(x0, wr_ref[r], preferred_element_type=F32)


def _tc_samplers(x0, macc2, n2g_col, seq_final, w_pn, w_pe, Wg_n, Wg_e, W_R):
    blk = lambda *shape: shape
    return pl.pallas_call(
        _samplers_body,
        grid=(NB,),
        in_specs=[
            pl.BlockSpec((BLK, D), lambda i: (i, 0)),
            pl.BlockSpec((2, BLK, D), lambda i: (0, i, 0)),
            pl.BlockSpec((BLK, 1), lambda i: (i, 0)),
            pl.BlockSpec((B, D), lambda i: (0, 0)),
            pl.BlockSpec((1, D), lambda i: (0, 0)),
            pl.BlockSpec((1, D), lambda i: (0, 0)),
            pl.BlockSpec((D, D), lambda i: (0, 0)),
            pl.BlockSpec((D, D), lambda i: (0, 0)),
            pl.BlockSpec((NR, D, D), lambda i: (0, 0, 0)),
        ],
        out_specs=[
            pl.BlockSpec((BLK, 1), lambda i: (i, 0)),
            pl.BlockSpec((BLK, D), lambda i: (i, 0)),
            pl.BlockSpec((BLK, B), lambda i: (i, 0)),
            pl.BlockSpec((NR, BLK, D), lambda i: (0, i, 0)),
            pl.BlockSpec((NR, BLK, D), lambda i: (0, i, 0)),
            pl.BlockSpec((B, D), lambda i: (0, 0)),
            pl.BlockSpec((B, D), lambda i: (0, 0)),
            pl.BlockSpec((B, D), lambda i: (0, 0)),
        ],
        out_shape=[
            jax.ShapeDtypeStruct((NP, 1), F32),
            jax.ShapeDtypeStruct((NP, D), F32),
            jax.ShapeDtypeStruct((NP, B), F32),
            jax.ShapeDtypeStruct((NR, NP, D), F32),
            jax.ShapeDtypeStruct((NR, NP, D), F32),
            jax.ShapeDtypeStruct((B, D), F32),
            jax.ShapeDtypeStruct((B, D), F32),
            jax.ShapeDtypeStruct((B, D), F32),
        ],
    )(x0, macc2, n2g_col, seq_final, w_pn, w_pe, Wg_n, Wg_e, W_R)


def _layer_body(emit_H, hn_ref, he_ref, an_ref, ae_ref, c_ref, rel_ref,
                n2g_ref, wl_ref, bl_ref, wr_ref, *outs):
    if emit_H:
        hn_o, he_o, H2n_ref, H2e_ref, pn_ref, pe_ref = outs
    else:
        hn_o, he_o, pn_ref, pe_ref = outs
    i = pl.program_id(0)
    relC = jnp.dot(c_ref[0] + c_ref[1], rel_ref[...], preferred_element_type=F32)
    aggn = an_ref[0] + an_ref[1] + relC
    agge = ae_ref[0] + ae_ref[1] + relC
    hn = jnp.maximum(
        jnp.dot(hn_ref[...] + aggn, wl_ref[...], preferred_element_type=F32)
        + bl_ref[...], 0.0)
    he = jnp.maximum(
        jnp.dot(he_ref[...] + agge, wl_ref[...], preferred_element_type=F32)
        + bl_ref[...], 0.0)
    hn_o[...] = hn
    he_o[...] = he
    onehot = (n2g_ref[...] == lax.broadcasted_iota(I32, (1, B), 1)).astype(F32)
    dimp = (((0,), (0,)), ((), ()))

    @pl.when(i == 0)
    def _():
        pn_ref[...] = jnp.zeros((B, D), F32)
        pe_ref[...] = jnp.zeros((B, D), F32)

    pn_ref[...] += lax.dot_general(onehot, hn, dimp, preferred_element_type=F32)
    pe_ref[...] += lax.dot_general(onehot, he, dimp, preferred_element_type=F32)
    if emit_H:
        for r in range(NR):
            H2n_ref[r] = jnp.dot(hn, wr_ref[r], preferred_element_type=F32)
            H2e_ref[r] = jnp.dot(he, wr_ref[r], preferred_element_type=F32)


def _tc_layer(hn, he, accn2, acce2, C2, rel_tab, n2g_col, W_l, b_l, W_R,
              emit_H):
    out_specs = [
        pl.BlockSpec((BLK, D), lambda i: (i, 0)),
        pl.BlockSpec((BLK, D), lambda i: (i, 0)),
    ]
    out_shape = [jax.ShapeDtypeStruct((NP, D), F32),
                 jax.ShapeDtypeStruct((NP, D), F32)]
    if emit_H:
        out_specs += [pl.BlockSpec((NR, BLK, D), lambda i: (0, i, 0))] * 2
        out_shape += [jax.ShapeDtypeStruct((NR, NP, D), F32)] * 2
    out_specs += [pl.BlockSpec((B, D), lambda i: (0, 0))] * 2
    out_shape += [jax.ShapeDtypeStruct((B, D), F32)] * 2
    return pl.pallas_call(
        functools.partial(_layer_body, emit_H),
        grid=(NB,),
        in_specs=[
            pl.BlockSpec((BLK, D), lambda i: (i, 0)),
            pl.BlockSpec((BLK, D), lambda i: (i, 0)),
            pl.BlockSpec((2, BLK, D), lambda i: (0, i, 0)),
            pl.BlockSpec((2, BLK, D), lambda i: (0, i, 0)),
            pl.BlockSpec((2, BLK, NR), lambda i: (0, i, 0)),
            pl.BlockSpec((NR, D), lambda i: (0, 0)),
            pl.BlockSpec((BLK, 1), lambda i: (i, 0)),
            pl.BlockSpec((D, D), lambda i: (0, 0)),
            pl.BlockSpec((1, D), lambda i: (0, 0)),
            pl.BlockSpec((NR, D, D), lambda i: (0, 0, 0)),
        ],
        out_specs=out_specs,
        out_shape=out_shape,
    )(hn, he, accn2, acce2, C2, rel_tab, n2g_col, W_l, b_l, W_R)


def _loss_body(p0n, p1n, p2n, p0e, p1e, p2e, cnt, loss_ref):
    counts = jnp.maximum(cnt[...], 1.0)
    x1 = jnp.concatenate([p0n[...] / counts, p1n[...] / counts,
                          p2n[...] / counts], axis=1)
    x2 = jnp.concatenate([p0e[...] / counts, p1e[...] / counts,
                          p2e[...] / counts], axis=1)
    n1 = jnp.sqrt(jnp.sum(x1 * x1, axis=1, keepdims=True))
    n2 = jnp.sqrt(jnp.sum(x2 * x2, axis=1, keepdims=True))
    dimn = (((1,), (1,)), ((), ()))
    sim = lax.dot_general(x1, x2, dimn, preferred_element_type=F32)
    nn = lax.dot_general(n1, n2, (((1,), (1,)), ((), ())),
                         preferred_element_type=F32)
    ea = jnp.exp(sim / nn / 0.5)
    eye = (lax.broadcasted_iota(I32, (B, B), 0)
           == lax.broadcasted_iota(I32, (B, B), 1)).astype(F32)
    pos = jnp.sum(ea * eye, axis=1, keepdims=True)
    rs = jnp.sum(ea, axis=1, keepdims=True)
    cs = jnp.sum(ea * eye, axis=0, keepdims=True)
    csf = jnp.sum(ea, axis=0, keepdims=True)
    la = -jnp.sum(jnp.log(pos / (rs - pos))) / B
    lb = -jnp.sum(jnp.log(cs / (csf - cs))) / B
    loss_ref[0, 0] = 0.5 * (la + lb)


def _tc_loss(p0n, p1n, p2n, p0e, p1e, p2e, cnt):
    full = pl.BlockSpec((B, D), lambda: (0, 0))
    return pl.pallas_call(
        _loss_body,
        in_specs=[full] * 7,
        out_specs=pl.BlockSpec(memory_space=pltpu.SMEM),
        out_shape=jax.ShapeDtypeStruct((1, 1), F32),
    )(p0n, p1n, p2n, p0e, p1e, p2e, cnt)


_MESH = plsc.VectorSubcoreMesh(core_axis_name="c", subcore_axis_name="s",
                               num_cores=NC, num_subcores=NS)
IPW = NP // NW


def _x0_body(tab, ids, out, idx_v, rows_v, sem):
    wid = lax.axis_index("s") * NC + lax.axis_index("c")
    base = wid * IPW

    def body(j, carry):
        off = base + j * CH
        pltpu.sync_copy(ids.at[pl.ds(off, CH)], idx_v)
        pltpu.async_copy(tab.at[idx_v], rows_v, sem).wait()
        pltpu.sync_copy(rows_v, out.at[pl.ds(off, CH)])
        return carry

    lax.fori_loop(0, IPW // CH, body, 0)


def _sc_x0gather(node_tab, ids_p):
    return pl.kernel(
        _x0_body,
        out_type=jax.ShapeDtypeStruct((NP, D), F32),
        mesh=_MESH,
        scratch_types=[pltpu.VMEM((CH,), I32), pltpu.VMEM((CH, D), F32),
                       pltpu.SemaphoreType.DMA],
    )(node_tab, ids_p)


def _zero_vmem_2d(ref, nrows):
    def body(i, carry):
        r = i // (D // L)
        c = i % (D // L)
        ref[r, pl.ds(c * L, L)] = jnp.zeros((L,), F32)
        return carry
    lax.fori_loop(0, nrows * (D // L), body, 0)


def _zero_vmem_1d(ref, n):
    def body(i, carry):
        ref[pl.ds(i * L, L)] = jnp.zeros((L,), F32)
        return carry
    lax.fori_loop(0, n // L, body, 0)


def _edge_epilogue(acc_sh, acc_out, cid, sid):
    plsc.subcore_barrier()
    for j in range(ROWS_PER_TILE // D):
        off = sid * ROWS_PER_TILE + j * D
        pltpu.sync_copy(acc_sh.at[pl.ds(off, D)], acc_out.at[cid, pl.ds(off, D)])


def _scale_rows(rows_v, get_pvec):
    def body(g, carry):
        pv = get_pvec(g)
        for j in range(L):
            w = jnp.full((L,), pv[j], F32)
            e = g * L + j
            for f in range(D // L):
                rows_v[e, pl.ds(f * L, L)] = rows_v[e, pl.ds(f * L, L)] * w
        return carry
    lax.fori_loop(0, ECH // L, body, 0)


NCHUNK_ROWS = E_PAD // ECH


def _make_edge_pass(nidx, ka, compute_p=False, preload_p=False):
    def body(*refs):
        it = iter(refs)
        tab = next(it); ipack = next(it)
        gef = next(it) if compute_p else None
        acc_out = next(it)
        pedge_out = next(it) if compute_p else None
        rows = (next(it), next(it))
        ibuf = (next(it), next(it), next(it), next(it))
        if compute_p:
            g1 = (next(it), next(it))
            g2 = (next(it), next(it))
            pbuf = (next(it), next(it))
        gsem = (next(it), next(it))
        ssem = (next(it), next(it))
        isem = (next(it), next(it), next(it), next(it))
        psem = (next(it), next(it)) if compute_p else None
        acc_sh = next(it)

        cid = lax.axis_index("c")
        sid = lax.axis_index("s")

        def zr(i, carry):
            r = i // (D // L)
            c = i % (D // L)
            rows[0][r, pl.ds(c * L, L)] = jnp.zeros((L,), F32)
            return carry

        lax.fori_loop(0, ECH * (D // L), zr, 0)
        for j in range(ROWS_PER_TILE // ECH):
            pltpu.sync_copy(
                rows[0], acc_sh.at[pl.ds(sid * ROWS_PER_TILE + j * ECH, ECH)])
        plsc.subcore_barrier()

        def issue_gather(b2, b3):
            pltpu.async_copy(tab.at[ibuf[b3].at[0]], rows[b2], gsem[b2])
            if compute_p:
                pltpu.async_copy(gef.at[ibuf[b3].at[2]], g1[b2], gsem[b2])
                pltpu.async_copy(gef.at[ibuf[b3].at[3]], g2[b2], gsem[b2])

        def wait_gather(b2, b3):
            pltpu.make_async_copy(tab.at[ibuf[b3].at[0]], rows[b2],
                                  gsem[b2]).wait()
            if compute_p:
                pltpu.make_async_copy(gef.at[ibuf[b3].at[2]], g1[b2],
                                      gsem[b2]).wait()
                pltpu.make_async_copy(gef.at[ibuf[b3].at[3]], g2[b2],
                                      gsem[b2]).wait()

        def wait_scatter(b2, b3):
            pltpu.make_async_copy(rows[b2], acc_sh.at[ibuf[b3].at[1]],
                                  ssem[b2]).wait()

        def pipeline(cbase, nch):
            def do_chunk(k, j):
                b2 = j % 2
                b3 = j % 4

                @pl.when(k >= 1)
                def _():
                    wait_scatter(1 - b2, (j + 3) % 4)

                @pl.when(k + 1 < nch)
                def _():
                    pltpu.make_async_copy(ipack.at[0], ibuf[(j + 1) % 4],
                                          isem[(j + 1) % 4]).wait()
                    issue_gather(1 - b2, (j + 1) % 4)

                @pl.when(k + 2 < nch)
                def _():
                    pltpu.async_copy(ipack.at[cbase + k + 2],
                                     ibuf[(j + 2) % 4], isem[(j + 2) % 4])

                wait_gather(b2, b3)
                if compute_p:
                    @pl.when(k >= 2)
                    def _():
                        pltpu.make_async_copy(pbuf[b2], pedge_out.at[0],
                                              psem[b2]).wait()

                    def grp(g, carry2):
                        a = g1[b2][pl.ds(g * L, L)]
                        c = g2[b2][pl.ds(g * L, L)]
                        pbuf[b2][pl.ds(g * L, L)] = 1.0 / (
                            1.0 + jnp.exp(-(a + c)))
                        return carry2

                    lax.fori_loop(0, ECH // L, grp, 0)
                    pltpu.async_copy(pbuf[b2], pedge_out.at[cbase + k],
                                     psem[b2])
                    _scale_rows(rows[b2], lambda g: pbuf[b2][pl.ds(g * L, L)])
                elif preload_p:
                    _scale_rows(
                        rows[b2],
                        lambda g: lax.bitcast_convert_type(
                            ibuf[b3][2, pl.ds(g * L, L)], F32))
                pltpu.async_copy(rows[b2], acc_sh.at[ibuf[b3].at[1]],
                                 ssem[b2], add=True)

            pltpu.sync_copy(ipack.at[cbase], ibuf[0])
            pltpu.async_copy(ipack.at[cbase + 1], ibuf[1], isem[1])
            issue_gather(0, 0)

            def group(g, carry):
                for j in range(4):
                    do_chunk(4 * g + j, j)
                return carry

            lax.fori_loop(0, nch // 4, group, 0)
            wait_scatter(1, 3)
            if compute_p:
                pltpu.make_async_copy(pbuf[0], pedge_out.at[0], psem[0]).wait()
                pltpu.make_async_copy(pbuf[1], pedge_out.at[0], psem[1]).wait()

        kb = NCHUNK_ROWS // NS - ka

        @pl.when(cid == 0)
        def _():
            pipeline(sid * ka, ka)

        @pl.when(cid == 1)
        def _():
            pipeline(NS * ka + sid * kb, kb)

        _edge_epilogue(acc_sh, acc_out, cid, sid)

    out_type = [jax.ShapeDtypeStruct((NC, NP, D), F32)]
    if compute_p:
        out_type.append(jax.ShapeDtypeStruct((NCHUNK_ROWS, ECH), F32))
    scr = [pltpu.VMEM((ECH, D), F32), pltpu.VMEM((ECH, D), F32)]
    scr += [pltpu.VMEM((nidx, ECH), I32)] * 4
    if compute_p:
        scr += [pltpu.VMEM((ECH,), F32)] * 6
    scr += [pltpu.SemaphoreType.DMA] * 8
    if compute_p:
        scr += [pltpu.SemaphoreType.DMA] * 2
    scr.append(pltpu.VMEM_SHARED((NP, D), F32))

    def run(*args):
        return pl.kernel(
            body,
            out_type=tuple(out_type) if len(out_type) > 1 else out_type[0],
            mesh=_MESH,
            scratch_types=scr,
        )(*args)

    return run


_sc_plain = _make_edge_pass(nidx=2, ka=148)
_sc_wpass1 = _make_edge_pass(nidx=4, ka=140, compute_p=True)
_sc_wpass2 = _make_edge_pass(nidx=3, ka=140, preload_p=True)


def _cpass_body(cidx2, c_out, cidx_all, ones_v, zc_v, csem, c_sh):
    cid = lax.axis_index("c")
    sid = lax.axis_index("s")
    wid = sid * NC + cid
    _zero_vmem_1d(zc_v, NPC // NS // 5)
    for j in range(5):
        pltpu.sync_copy(
            zc_v, c_sh.at[pl.ds(sid * (NPC // NS) + j * (NPC // NS // 5),
                                NPC // NS // 5)])

    def ones_init(i, carry):
        ones_v[pl.ds(i * L, L)] = jnp.ones((L,), F32)
        return carry

    lax.fori_loop(0, ECH // L, ones_init, 0)
    pltpu.sync_copy(cidx2.at[pl.ds(wid * NCH2, NCH2)], cidx_all)
    plsc.subcore_barrier()

    def chunk(k, carry):
        @pl.when(k >= 1)
        def _():
            pltpu.make_async_copy(ones_v, c_sh.at[cidx_all.at[0]],
                                  csem).wait()

        pltpu.async_copy(ones_v, c_sh.at[cidx_all.at[k]], csem, add=True)
        return carry

    lax.fori_loop(0, NCH2, chunk, 0)
    pltpu.make_async_copy(ones_v, c_sh.at[cidx_all.at[0]], csem).wait()
    plsc.subcore_barrier()
    coff = sid * (NPC // NS)
    pltpu.sync_copy(c_sh.at[pl.ds(coff, NPC // NS)],
                    c_out.at[cid, pl.ds(coff, NPC // NS)])


def _sc_cpass(cidx2):
    return pl.kernel(
        _cpass_body,
        out_type=jax.ShapeDtypeStruct((NC, NPC), F32),
        mesh=_MESH,
        scratch_types=[
            pltpu.VMEM((NCH2, ECH), I32),
            pltpu.VMEM((ECH,), F32),
            pltpu.VMEM((NPC // NS // 5,), F32),
            pltpu.SemaphoreType.DMA,
            pltpu.VMEM_SHARED((NPC,), F32),
        ],
    )(cidx2)


def kernel(x_batch, s_batch, s_batch_dim2, batch_mask, node_ids, edge_index,
           edge_type, node2graph, edgebindex, ehrcode_W, node_tab, rel_tab,
           W_R, gru_Wx, gru_Wh, gru_b, att_w, Wg_n, w_pn, Wg_e, w_pe,
           L1_W, L1_b, L2_W, L2_b):
    src = edge_index[0]
    dst = edge_index[1]
    ids_p = jnp.concatenate([node_ids.astype(I32),
                             jnp.full((NP - N,), N, I32)])
    n2g_col = jnp.concatenate([node2graph.astype(I32),
                               jnp.full((NP - N,), B, I32)]).reshape(NP, 1)

    x_tbc = jnp.transpose(x_batch, (1, 0, 2))
    seq_final, alpha = _tc_front(x_tbc, batch_mask, ehrcode_W, gru_Wx,
                                 gru_Wh, gru_b.reshape(1, 3 * D),
                                 att_w.reshape(1, D))

    npad = E_PAD - E
    srcp = jnp.concatenate([src.astype(I32), jnp.full((npad,), N, I32)])
    dstp = jnp.concatenate([dst.astype(I32), jnp.full((npad,), NP - 1, I32)])
    typp = jnp.concatenate([edge_type.astype(I32), jnp.zeros((npad,), I32)])
    ebp = jnp.concatenate([edgebindex.astype(I32), jnp.zeros((npad,), I32)])
    e2 = (E_PAD // D, D)
    srcp = srcp.reshape(e2)
    dstp = dstp.reshape(e2)
    gidx2, cidx2, idx12, idx22 = _tc_edgeidx(
        srcp, dstp, typp.reshape(e2), ebp.reshape(e2))

    ip_m = jnp.stack([srcp, dstp], axis=1)
    ip_g = jnp.stack([gidx2, dstp], axis=1)
    ip_w1 = jnp.stack([gidx2, dstp, idx12, idx22], axis=1)

    x0 = _sc_x0gather(node_tab, ids_p)
    macc2 = _sc_plain(x0, ip_m)
    C2 = _sc_cpass(cidx2).reshape(NC, NP, NR)

    (pn_col, view_x, Ge, H1n, H1e, p0n, p0e, cnt) = _tc_samplers(
        x0, macc2, n2g_col, seq_final, w_pn.reshape(1, D),
        w_pe.reshape(1, D), Wg_n, Wg_e, W_R)

    acc1n = _sc_plain(H1n.reshape(NR * NP, D), ip_g)
    acc1e, pedge2 = _sc_wpass1(H1e.reshape(NR * NP, D), ip_w1,
                               Ge.reshape(NP * B))
    p_edge = pedge2.reshape(E_PAD)[:E]

    h1n, h1e, H2n, H2e, p1n, p1e = _tc_layer(
        view_x, x0, acc1n, acc1e, C2, rel_tab, n2g_col, L1_W,
        L1_b.reshape(1, D), W_R, emit_H=True)

    acc2n = _sc_plain(H2n.reshape(NR * NP, D), ip_g)
    ip_w2 = jnp.stack(
        [gidx2, dstp, lax.bitcast_convert_type(pedge2, I32)], axis=1)
    acc2e = _sc_wpass2(H2e.reshape(NR * NP, D), ip_w2)

    h2n, h2e, p2n, p2e = _tc_layer(
        h1n, h1e, acc2n, acc2e, C2, rel_tab, n2g_col, L2_W,
        L2_b.reshape(1, D), W_R, emit_H=False)

    loss = _tc_loss(p0n, p1n, p2n, p0e, p1e, p2e, cnt)[0, 0]

    p_node = pn_col.reshape(NP)[:N]
    return (loss, p_node, p_node, p_edge, p_edge, seq_final, alpha)
